# baseline (speedup 1.0000x reference)
import jax
import jax.numpy as jnp
from jax.experimental import pallas as pl
from jax.experimental.pallas import tpu as pltpu

_BEV_EXTENT = (-32.0, -32.0, 32.0, 32.0)
_EPS = 1e-8
_Q = 512


def _body(qsrc_ref, tgtT_ref, fsrc_ref, ftgt_ref, sflow_ref, misc_ref,
          trafo_ref, out_ref):
    g = pl.program_id(0)
    q = pl.program_id(1)

    p = qsrc_ref[0]
    fsrc = fsrc_ref[0]
    w = p + fsrc

    tT = tgtT_ref[0]
    wn = jnp.sum(w * w, axis=1, keepdims=True)
    tn = jnp.sum(tT * tT, axis=0, keepdims=True)
    gram = jax.lax.dot_general(w, tT, (((1,), (0,)), ((), ())),
                               preferred_element_type=jnp.float32)
    u = tn - 2.0 * gram
    umin = jnp.min(u, axis=1, keepdims=True)
    nn_d2 = jnp.maximum(wn + umin, 0.0)
    onehot = (u == umin).astype(jnp.float32)
    fnn = jax.lax.dot_general(onehot, ftgt_ref[0], (((1,), (0,)), ((), ())),
                              preferred_element_type=jnp.float32)

    x0, y0, x1, y1 = _BEV_EXTENT
    wx = w[:, 0:1]
    wy = w[:, 1:2]
    in_bev = ((wx >= x0) & (wx <= x1) & (wy >= y0) & (wy <= y1)
              ).astype(jnp.float32)

    knn_num = jnp.sum(in_bev * nn_d2)
    knn_den = jnp.sum(in_bev)
    opp = fsrc + fnn
    opp_err = jnp.sum(opp * opp, axis=1, keepdims=True)
    opp_num = jnp.sum(in_bev * opp_err)

    trafo = trafo_ref[g]
    rot = trafo[:3, :3]
    trans = trafo[:3, 3]
    rp = jax.lax.dot_general(p, rot, (((1,), (1,)), ((), ())),
                             preferred_element_type=jnp.float32)
    trafo_flow = rp + trans[None, :] - p
    serr = sflow_ref[0] - trafo_flow
    serr2 = jnp.sum(serr * serr, axis=1, keepdims=True)
    staticness = misc_ref[0][:, 0:1]
    static_num = jnp.sum(staticness * serr2)
    static_den = jnp.sum(staticness)

    dis = misc_ref[0][:, 1:2]
    valid = (dis == dis).astype(jnp.float32)
    occ_sum = jnp.sum(jnp.where(dis == dis, dis, 0.0))
    occ_cnt = jnp.sum(valid)

    nb = trafo_ref.shape[0] // 2
    eye = jnp.eye(4, dtype=jnp.float32)
    sse = jnp.float32(0.0)
    for b in range(nb):
        comp = jnp.dot(trafo_ref[b], trafo_ref[nb + b],
                       preferred_element_type=jnp.float32)
        dlt = comp - eye
        sse = sse + jnp.sum(dlt * dlt)
    first = jnp.logical_and(g == 0, q == 0).astype(jnp.float32)
    sse = sse * first

    slots = jax.lax.broadcasted_iota(jnp.int32, (1, 1, 8), 2)
    vals = [knn_num, knn_den, opp_num, static_num, static_den, occ_sum,
            occ_cnt, sse]
    row = jnp.zeros((1, 1, 8), jnp.float32)
    for k, v in enumerate(vals):
        row = row + jnp.where(slots == k, v, 0.0)

    @pl.when(q == 0)
    def _():
        out_ref[...] = row

    @pl.when(q != 0)
    def _():
        out_ref[...] += row


def kernel(pc0, pc1, fw_aggregated_flow, bw_aggregated_flow, fw_static_flow,
           bw_static_flow, fw_static_aggr_trafo, bw_static_aggr_trafo,
           fw_staticness, bw_staticness, fw_disappearing, bw_disappearing):
    B, N, _ = pc0.shape
    G = 2 * B

    qsrc = jnp.concatenate([pc0, pc1], axis=0)
    tgtT = jnp.concatenate([pc1, pc0], axis=0).transpose(0, 2, 1)
    fsrc = jnp.concatenate([fw_aggregated_flow, bw_aggregated_flow], axis=0)
    ftgt = jnp.concatenate([bw_aggregated_flow, fw_aggregated_flow], axis=0)
    sflow = jnp.concatenate([fw_static_flow, bw_static_flow], axis=0)
    misc = jnp.stack([jnp.concatenate([fw_staticness, bw_staticness], axis=0),
                      jnp.concatenate([fw_disappearing, bw_disappearing],
                                      axis=0)], axis=-1)
    trafos = jnp.concatenate([fw_static_aggr_trafo, bw_static_aggr_trafo],
                             axis=0)

    nq = N // _Q
    out = pl.pallas_call(
        _body,
        grid=(G, nq),
        in_specs=[
            pl.BlockSpec((1, _Q, 3), lambda g, q: (g, q, 0)),
            pl.BlockSpec((1, 3, N), lambda g, q: (g, 0, 0)),
            pl.BlockSpec((1, _Q, 3), lambda g, q: (g, q, 0)),
            pl.BlockSpec((1, N, 3), lambda g, q: (g, 0, 0)),
            pl.BlockSpec((1, _Q, 3), lambda g, q: (g, q, 0)),
            pl.BlockSpec((1, _Q, 2), lambda g, q: (g, q, 0)),
            pl.BlockSpec((G, 4, 4), lambda g, q: (0, 0, 0)),
        ],
        out_specs=pl.BlockSpec((1, 1, 8), lambda g, q: (g, 0, 0)),
        out_shape=jax.ShapeDtypeStruct((G, 1, 8), jnp.float32),
        compiler_params=pltpu.CompilerParams(
            dimension_semantics=("parallel", "arbitrary")),
    )(qsrc, tgtT, fsrc, ftgt, sflow, misc, trafos)

    out = out.reshape(G, 8)
    fw = out[:B]
    bw = out[B:]
    eps = jnp.float32(_EPS)

    def seg(rows):
        s = jnp.sum(rows, axis=0)
        den = s[1] + eps
        return s[0] / den, s[2] / den, s[3] / (s[4] + eps)

    fw_fl, fw_opp, fw_static = seg(fw)
    bw_fl, bw_opp, bw_static = seg(bw)
    flow_loss = 0.5 * (fw_fl + bw_fl)
    opposite_flow_loss = 0.5 * (fw_opp + bw_opp)
    static_flow_loss = 0.5 * (fw_static + bw_static)
    occlusion_loss = jnp.sum(out[:, 5]) / (jnp.sum(out[:, 6]) + eps)
    trafo_loss = jnp.sum(out[:, 7]) / (B * 16.0)

    total = (static_flow_loss + trafo_loss + 0.1 * occlusion_loss
             + flow_loss + opposite_flow_loss)
    return total


# speedup vs baseline: 1.0525x; 1.0525x over previous
import jax
import jax.numpy as jnp
from jax.experimental import pallas as pl
from jax.experimental.pallas import tpu as pltpu

_BEV_EXTENT = (-32.0, -32.0, 32.0, 32.0)
_EPS = 1e-8
_Q = 1024


def _body(qsrc_ref, tgtT_ref, fsrc_ref, ftgt_ref, sflow_ref, misc_ref,
          trafo_ref, out_ref):
    g = pl.program_id(0)
    q = pl.program_id(1)

    p = qsrc_ref[0]
    fsrc = fsrc_ref[0]
    w = p + fsrc

    tT = tgtT_ref[0]
    wn = jnp.sum(w * w, axis=1, keepdims=True)
    tn = jnp.sum(tT * tT, axis=0, keepdims=True)
    gram = jax.lax.dot_general(w, tT, (((1,), (0,)), ((), ())),
                               preferred_element_type=jnp.float32)
    u = tn - 2.0 * gram
    umin = jnp.min(u, axis=1, keepdims=True)
    nn_d2 = jnp.maximum(wn + umin, 0.0)
    onehot = (u == umin).astype(jnp.float32)
    fnn = jax.lax.dot_general(onehot, ftgt_ref[0], (((1,), (0,)), ((), ())),
                              preferred_element_type=jnp.float32)

    x0, y0, x1, y1 = _BEV_EXTENT
    wx = w[:, 0:1]
    wy = w[:, 1:2]
    in_bev = ((wx >= x0) & (wx <= x1) & (wy >= y0) & (wy <= y1)
              ).astype(jnp.float32)

    knn_num = jnp.sum(in_bev * nn_d2)
    knn_den = jnp.sum(in_bev)
    opp = fsrc + fnn
    opp_err = jnp.sum(opp * opp, axis=1, keepdims=True)
    opp_num = jnp.sum(in_bev * opp_err)

    trafo = trafo_ref[g]
    rot = trafo[:3, :3]
    trans = trafo[:3, 3]
    rp = jax.lax.dot_general(p, rot, (((1,), (1,)), ((), ())),
                             preferred_element_type=jnp.float32)
    trafo_flow = rp + trans[None, :] - p
    serr = sflow_ref[0] - trafo_flow
    serr2 = jnp.sum(serr * serr, axis=1, keepdims=True)
    staticness = misc_ref[0][:, 0:1]
    static_num = jnp.sum(staticness * serr2)
    static_den = jnp.sum(staticness)

    dis = misc_ref[0][:, 1:2]
    valid = (dis == dis).astype(jnp.float32)
    occ_sum = jnp.sum(jnp.where(dis == dis, dis, 0.0))
    occ_cnt = jnp.sum(valid)

    nb = trafo_ref.shape[0] // 2
    eye = jnp.eye(4, dtype=jnp.float32)
    sse = jnp.float32(0.0)
    for b in range(nb):
        comp = jnp.dot(trafo_ref[b], trafo_ref[nb + b],
                       preferred_element_type=jnp.float32)
        dlt = comp - eye
        sse = sse + jnp.sum(dlt * dlt)
    first = jnp.logical_and(g == 0, q == 0).astype(jnp.float32)
    sse = sse * first

    slots = jax.lax.broadcasted_iota(jnp.int32, (1, 1, 8), 2)
    vals = [knn_num, knn_den, opp_num, static_num, static_den, occ_sum,
            occ_cnt, sse]
    row = jnp.zeros((1, 1, 8), jnp.float32)
    for k, v in enumerate(vals):
        row = row + jnp.where(slots == k, v, 0.0)

    @pl.when(q == 0)
    def _():
        out_ref[...] = row

    @pl.when(q != 0)
    def _():
        out_ref[...] += row


def kernel(pc0, pc1, fw_aggregated_flow, bw_aggregated_flow, fw_static_flow,
           bw_static_flow, fw_static_aggr_trafo, bw_static_aggr_trafo,
           fw_staticness, bw_staticness, fw_disappearing, bw_disappearing):
    B, N, _ = pc0.shape
    G = 2 * B

    qsrc = jnp.concatenate([pc0, pc1], axis=0)
    tgtT = jnp.concatenate([pc1, pc0], axis=0).transpose(0, 2, 1)
    fsrc = jnp.concatenate([fw_aggregated_flow, bw_aggregated_flow], axis=0)
    ftgt = jnp.concatenate([bw_aggregated_flow, fw_aggregated_flow], axis=0)
    sflow = jnp.concatenate([fw_static_flow, bw_static_flow], axis=0)
    misc = jnp.stack([jnp.concatenate([fw_staticness, bw_staticness], axis=0),
                      jnp.concatenate([fw_disappearing, bw_disappearing],
                                      axis=0)], axis=-1)
    trafos = jnp.concatenate([fw_static_aggr_trafo, bw_static_aggr_trafo],
                             axis=0)

    nq = N // _Q
    out = pl.pallas_call(
        _body,
        grid=(G, nq),
        in_specs=[
            pl.BlockSpec((1, _Q, 3), lambda g, q: (g, q, 0)),
            pl.BlockSpec((1, 3, N), lambda g, q: (g, 0, 0)),
            pl.BlockSpec((1, _Q, 3), lambda g, q: (g, q, 0)),
            pl.BlockSpec((1, N, 3), lambda g, q: (g, 0, 0)),
            pl.BlockSpec((1, _Q, 3), lambda g, q: (g, q, 0)),
            pl.BlockSpec((1, _Q, 2), lambda g, q: (g, q, 0)),
            pl.BlockSpec((G, 4, 4), lambda g, q: (0, 0, 0)),
        ],
        out_specs=pl.BlockSpec((1, 1, 8), lambda g, q: (g, 0, 0)),
        out_shape=jax.ShapeDtypeStruct((G, 1, 8), jnp.float32),
        compiler_params=pltpu.CompilerParams(
            dimension_semantics=("parallel", "arbitrary")),
    )(qsrc, tgtT, fsrc, ftgt, sflow, misc, trafos)

    out = out.reshape(G, 8)
    fw = out[:B]
    bw = out[B:]
    eps = jnp.float32(_EPS)

    def seg(rows):
        s = jnp.sum(rows, axis=0)
        den = s[1] + eps
        return s[0] / den, s[2] / den, s[3] / (s[4] + eps)

    fw_fl, fw_opp, fw_static = seg(fw)
    bw_fl, bw_opp, bw_static = seg(bw)
    flow_loss = 0.5 * (fw_fl + bw_fl)
    opposite_flow_loss = 0.5 * (fw_opp + bw_opp)
    static_flow_loss = 0.5 * (fw_static + bw_static)
    occlusion_loss = jnp.sum(out[:, 5]) / (jnp.sum(out[:, 6]) + eps)
    trafo_loss = jnp.sum(out[:, 7]) / (B * 16.0)

    total = (static_flow_loss + trafo_loss + 0.1 * occlusion_loss
             + flow_loss + opposite_flow_loss)
    return total


# speedup vs baseline: 1.1569x; 1.0993x over previous
import jax
import jax.numpy as jnp
from jax.experimental import pallas as pl
from jax.experimental.pallas import tpu as pltpu

_BEV_EXTENT = (-32.0, -32.0, 32.0, 32.0)
_EPS = 1e-8
_Q = 1024


def _body(qsrc_ref, fsrc_ref, taug_ref, ftgt_ref, sflow_ref, misc_ref,
          trafo_ref, out_ref):
    g = pl.program_id(0)
    q = pl.program_id(1)

    p4 = qsrc_ref[0]
    f4 = fsrc_ref[0]
    w4 = p4 + f4
    p = p4[:, 0:3]
    fsrc = f4[:, 0:3]
    w = w4[:, 0:3]

    u = jax.lax.dot_general(w4, taug_ref[0], (((1,), (0,)), ((), ())),
                            preferred_element_type=jnp.float32)
    umin = jnp.min(u, axis=1, keepdims=True)
    wn = jnp.sum(w * w, axis=1, keepdims=True)
    nn_d2 = jnp.maximum(wn + umin, 0.0)
    onehot = (u == umin).astype(jnp.float32)
    fnn = jax.lax.dot_general(onehot, ftgt_ref[0], (((1,), (0,)), ((), ())),
                              preferred_element_type=jnp.float32)

    x0, y0, x1, y1 = _BEV_EXTENT
    wx = w4[:, 0:1]
    wy = w4[:, 1:2]
    in_bev = ((wx >= x0) & (wx <= x1) & (wy >= y0) & (wy <= y1)
              ).astype(jnp.float32)

    knn_num = jnp.sum(in_bev * nn_d2)
    knn_den = jnp.sum(in_bev)
    opp = fsrc + fnn
    opp_err = jnp.sum(opp * opp, axis=1, keepdims=True)
    opp_num = jnp.sum(in_bev * opp_err)

    trafo = trafo_ref[g]
    rot = trafo[:3, :3]
    trans = trafo[:3, 3]
    rp = jax.lax.dot_general(p, rot, (((1,), (1,)), ((), ())),
                             preferred_element_type=jnp.float32)
    trafo_flow = rp + trans[None, :] - p
    serr = sflow_ref[0] - trafo_flow
    serr2 = jnp.sum(serr * serr, axis=1, keepdims=True)
    staticness = misc_ref[0][:, 0:1]
    static_num = jnp.sum(staticness * serr2)
    static_den = jnp.sum(staticness)

    dis = misc_ref[0][:, 1:2]
    valid = (dis == dis).astype(jnp.float32)
    occ_sum = jnp.sum(jnp.where(dis == dis, dis, 0.0))
    occ_cnt = jnp.sum(valid)

    nb = trafo_ref.shape[0] // 2
    eye = jnp.eye(4, dtype=jnp.float32)
    sse = jnp.float32(0.0)
    for b in range(nb):
        comp = jnp.dot(trafo_ref[b], trafo_ref[nb + b],
                       preferred_element_type=jnp.float32)
        dlt = comp - eye
        sse = sse + jnp.sum(dlt * dlt)
    first = jnp.logical_and(g == 0, q == 0).astype(jnp.float32)
    sse = sse * first

    slots = jax.lax.broadcasted_iota(jnp.int32, (1, 1, 8), 2)
    vals = [knn_num, knn_den, opp_num, static_num, static_den, occ_sum,
            occ_cnt, sse]
    row = jnp.zeros((1, 1, 8), jnp.float32)
    for k, v in enumerate(vals):
        row = row + jnp.where(slots == k, v, 0.0)

    @pl.when(q == 0)
    def _():
        out_ref[...] = row

    @pl.when(q != 0)
    def _():
        out_ref[...] += row


def kernel(pc0, pc1, fw_aggregated_flow, bw_aggregated_flow, fw_static_flow,
           bw_static_flow, fw_static_aggr_trafo, bw_static_aggr_trafo,
           fw_staticness, bw_staticness, fw_disappearing, bw_disappearing):
    B, N, _ = pc0.shape
    G = 2 * B

    ones = jnp.ones((B, N, 1), jnp.float32)
    zeros = jnp.zeros((B, N, 1), jnp.float32)
    qsrc = jnp.concatenate(
        [jnp.concatenate([pc0, ones], axis=-1),
         jnp.concatenate([pc1, ones], axis=-1)], axis=0)
    fsrc = jnp.concatenate(
        [jnp.concatenate([fw_aggregated_flow, zeros], axis=-1),
         jnp.concatenate([bw_aggregated_flow, zeros], axis=-1)], axis=0)

    pc0T = pc0.transpose(0, 2, 1)
    pc1T = pc1.transpose(0, 2, 1)
    tn0 = jnp.sum(pc0 * pc0, axis=-1)[:, None, :]
    tn1 = jnp.sum(pc1 * pc1, axis=-1)[:, None, :]
    taug = jnp.concatenate(
        [jnp.concatenate([-2.0 * pc1T, tn1], axis=1),
         jnp.concatenate([-2.0 * pc0T, tn0], axis=1)], axis=0)

    ftgt = jnp.concatenate([bw_aggregated_flow, fw_aggregated_flow], axis=0)
    sflow = jnp.concatenate([fw_static_flow, bw_static_flow], axis=0)
    misc = jnp.stack([jnp.concatenate([fw_staticness, bw_staticness], axis=0),
                      jnp.concatenate([fw_disappearing, bw_disappearing],
                                      axis=0)], axis=-1)
    trafos = jnp.concatenate([fw_static_aggr_trafo, bw_static_aggr_trafo],
                             axis=0)

    nq = N // _Q
    out = pl.pallas_call(
        _body,
        grid=(G, nq),
        in_specs=[
            pl.BlockSpec((1, _Q, 4), lambda g, q: (g, q, 0)),
            pl.BlockSpec((1, _Q, 4), lambda g, q: (g, q, 0)),
            pl.BlockSpec((1, 4, N), lambda g, q: (g, 0, 0)),
            pl.BlockSpec((1, N, 3), lambda g, q: (g, 0, 0)),
            pl.BlockSpec((1, _Q, 3), lambda g, q: (g, q, 0)),
            pl.BlockSpec((1, _Q, 2), lambda g, q: (g, q, 0)),
            pl.BlockSpec((G, 4, 4), lambda g, q: (0, 0, 0)),
        ],
        out_specs=pl.BlockSpec((1, 1, 8), lambda g, q: (g, 0, 0)),
        out_shape=jax.ShapeDtypeStruct((G, 1, 8), jnp.float32),
        compiler_params=pltpu.CompilerParams(
            dimension_semantics=("parallel", "arbitrary")),
    )(qsrc, fsrc, taug, ftgt, sflow, misc, trafos)

    out = out.reshape(G, 8)
    fw = out[:B]
    bw = out[B:]
    eps = jnp.float32(_EPS)

    def seg(rows):
        s = jnp.sum(rows, axis=0)
        den = s[1] + eps
        return s[0] / den, s[2] / den, s[3] / (s[4] + eps)

    fw_fl, fw_opp, fw_static = seg(fw)
    bw_fl, bw_opp, bw_static = seg(bw)
    flow_loss = 0.5 * (fw_fl + bw_fl)
    opposite_flow_loss = 0.5 * (fw_opp + bw_opp)
    static_flow_loss = 0.5 * (fw_static + bw_static)
    occlusion_loss = jnp.sum(out[:, 5]) / (jnp.sum(out[:, 6]) + eps)
    trafo_loss = jnp.sum(out[:, 7]) / (B * 16.0)

    total = (static_flow_loss + trafo_loss + 0.1 * occlusion_loss
             + flow_loss + opposite_flow_loss)
    return total
